# Initial kernel scaffold; baseline (speedup 1.0000x reference)
import jax
import jax.numpy as jnp
from jax.experimental import pallas as pl


def kernel(x, router_w, router_b, gate_w, up_w, down_w):
    raise NotImplementedError("write your pallas kernel here")


# speedup vs baseline: 1.9603x; 1.9603x over previous
import functools

import jax
import jax.numpy as jnp
from jax.experimental import pallas as pl
from jax.experimental.pallas import tpu as pltpu

E = 8
K = 2
BLK = 256
DI_T = 1024


def _ffn_body(be_ref, xs_ref, gw_ref, uw_ref, dw_ref, out_ref):
    j = pl.program_id(1)
    x = xs_ref[...]
    g = jax.lax.dot_general(x, gw_ref[0], (((1,), (1,)), ((), ())),
                            preferred_element_type=jnp.float32)
    u = jax.lax.dot_general(x, uw_ref[0], (((1,), (1,)), ((), ())),
                            preferred_element_type=jnp.float32)
    h = g * jax.nn.sigmoid(g) * u
    o = jax.lax.dot_general(h, dw_ref[0], (((1,), (1,)), ((), ())),
                            preferred_element_type=jnp.float32)

    @pl.when(j == 0)
    def _():
        out_ref[...] = o

    @pl.when(j > 0)
    def _():
        out_ref[...] += o


def _grouped_ffn(xs, gate_w, up_w, down_w, block_expert):
    pt, d = xs.shape
    e, di, _ = gate_w.shape
    nb = pt // BLK
    it = di // DI_T
    grid_spec = pltpu.PrefetchScalarGridSpec(
        num_scalar_prefetch=1,
        grid=(nb, it),
        in_specs=[
            pl.BlockSpec((BLK, d), lambda i, j, be: (i, 0)),
            pl.BlockSpec((1, DI_T, d), lambda i, j, be: (be[i], j, 0)),
            pl.BlockSpec((1, DI_T, d), lambda i, j, be: (be[i], j, 0)),
            pl.BlockSpec((1, d, DI_T), lambda i, j, be: (be[i], 0, j)),
        ],
        out_specs=pl.BlockSpec((BLK, d), lambda i, j, be: (i, 0)),
    )
    return pl.pallas_call(
        _ffn_body,
        grid_spec=grid_spec,
        out_shape=jax.ShapeDtypeStruct((pt, d), jnp.float32),
    )(block_expert, xs, gate_w, up_w, down_w)


def kernel(x, router_w, router_b, gate_w, up_w, down_w):
    b, s, d = x.shape
    t = b * s
    tk = t * K
    pt = tk + E * BLK
    nb = pt // BLK

    flat_x = x.reshape(t, d)

    logits = flat_x @ router_w.T + router_b
    probs = jax.nn.softmax(logits, axis=-1)
    tk_p, tk_i = jax.lax.top_k(probs, K)
    tk_p = tk_p / jnp.sum(tk_p, axis=-1, keepdims=True)

    e_flat = tk_i.reshape(-1).astype(jnp.int32)

    order = jnp.argsort(e_flat, stable=True)
    es = e_flat[order]
    counts = jnp.sum(e_flat[:, None] == jnp.arange(E)[None, :], axis=0)
    padded = ((counts + BLK - 1) // BLK) * BLK
    pend = jnp.cumsum(padded)
    poff = pend - padded
    uoff = jnp.cumsum(counts) - counts
    dest_sorted = poff[es] + (jnp.arange(tk) - uoff[es])

    src = jnp.zeros((pt,), jnp.int32).at[dest_sorted].set(
        (order // K).astype(jnp.int32))
    dest_by_pair = jnp.zeros((tk,), jnp.int32).at[order].set(
        dest_sorted.astype(jnp.int32))

    block_expert = jnp.minimum(
        jnp.searchsorted(pend, jnp.arange(nb, dtype=jnp.int32) * BLK,
                         side='right'),
        E - 1).astype(jnp.int32)

    xs = flat_x[src]

    outs = _grouped_ffn(xs, gate_w, up_w, down_w, block_expert)

    gathered = outs[dest_by_pair.reshape(t, K)]
    y = jnp.sum(gathered * tk_p.reshape(t, K)[..., None], axis=1)
    return y.reshape(b, s, d)


# speedup vs baseline: 2.3084x; 1.1776x over previous
import functools

import jax
import jax.numpy as jnp
from jax.experimental import pallas as pl
from jax.experimental.pallas import tpu as pltpu

E = 8
K = 2
BLK = 256
DI_T = 1024
TB = 512
PB = TB * K


def _router_body(x_ref, rw_ref, rb_ref, e_ref, r_ref, w_ref, cnt_ref, carry):
    c = pl.program_id(0)

    @pl.when(c == 0)
    def _():
        carry[...] = jnp.zeros_like(carry)

    lg = jax.lax.dot_general(rw_ref[...], x_ref[...], (((1,), (1,)), ((), ())),
                             preferred_element_type=jnp.float32)
    lg = lg + rb_ref[...].reshape(E, 1)
    iota = jax.lax.broadcasted_iota(jnp.int32, (E, TB), 0)
    m1 = jnp.max(lg, axis=0, keepdims=True)
    i1 = jnp.min(jnp.where(lg == m1, iota, E), axis=0, keepdims=True)
    masked = jnp.where(iota == i1, -jnp.inf, lg)
    m2 = jnp.max(masked, axis=0, keepdims=True)
    i2 = jnp.min(jnp.where((masked == m2) & (iota != i1), iota, E),
                 axis=0, keepdims=True)
    e21 = jnp.exp(m2 - m1)
    s = 1.0 + e21
    w1 = 1.0 / s
    w2 = e21 / s

    oh = jnp.concatenate([(iota == i1), (iota == i2)],
                         axis=1).astype(jnp.float32)
    pr = jax.lax.broadcasted_iota(jnp.int32, (PB, PB), 0)
    pc = jax.lax.broadcasted_iota(jnp.int32, (PB, PB), 1)
    ltu = (pr < pc).astype(jnp.float32)
    cum = jax.lax.dot_general(oh, ltu, (((1,), (0,)), ((), ())),
                              preferred_element_type=jnp.float32)
    rank = (jnp.sum(oh * cum, axis=0, keepdims=True)
            + jax.lax.dot_general(carry[...], oh, (((1,), (0,)), ((), ())),
                                  preferred_element_type=jnp.float32))

    evec = jnp.concatenate([i1, i2], axis=1)
    wvec = jnp.concatenate([w1, w2], axis=1)
    e_ref[...] = evec.reshape(PB)
    r_ref[...] = rank.astype(jnp.int32).reshape(PB)
    w_ref[...] = wvec.reshape(PB)
    carry[...] += jax.lax.dot_general(
        jnp.ones((1, PB), jnp.float32), oh, (((1,), (1,)), ((), ())),
        preferred_element_type=jnp.float32)
    cnt_ref[...] = carry[...].astype(jnp.int32).reshape(E)


def _router(flat_x, router_w, router_b):
    t, d = flat_x.shape
    nc = t // TB
    tk = t * K
    grid_spec = pltpu.PrefetchScalarGridSpec(
        num_scalar_prefetch=0,
        grid=(nc,),
        in_specs=[
            pl.BlockSpec((TB, d), lambda c: (c, 0)),
            pl.BlockSpec((E, d), lambda c: (0, 0)),
            pl.BlockSpec((1, E), lambda c: (0, 0)),
        ],
        out_specs=[
            pl.BlockSpec((PB,), lambda c: (c,)),
            pl.BlockSpec((PB,), lambda c: (c,)),
            pl.BlockSpec((PB,), lambda c: (c,)),
            pl.BlockSpec((E,), lambda c: (0,)),
        ],
        scratch_shapes=[pltpu.VMEM((1, E), jnp.float32)],
    )
    return pl.pallas_call(
        _router_body,
        grid_spec=grid_spec,
        out_shape=[
            jax.ShapeDtypeStruct((tk,), jnp.int32),
            jax.ShapeDtypeStruct((tk,), jnp.int32),
            jax.ShapeDtypeStruct((tk,), jnp.float32),
            jax.ShapeDtypeStruct((E,), jnp.int32),
        ],
    )(flat_x, router_w, router_b.reshape(1, E))


def _ffn_body(be_ref, xs_ref, gw_ref, uw_ref, dw_ref, out_ref):
    j = pl.program_id(1)
    x = xs_ref[...]
    g = jax.lax.dot_general(x, gw_ref[0], (((1,), (1,)), ((), ())),
                            preferred_element_type=jnp.float32)
    u = jax.lax.dot_general(x, uw_ref[0], (((1,), (1,)), ((), ())),
                            preferred_element_type=jnp.float32)
    h = g * jax.nn.sigmoid(g) * u
    o = jax.lax.dot_general(h, dw_ref[0], (((1,), (1,)), ((), ())),
                            preferred_element_type=jnp.float32)

    @pl.when(j == 0)
    def _():
        out_ref[...] = o

    @pl.when(j > 0)
    def _():
        out_ref[...] += o


def _grouped_ffn(xs, gate_w, up_w, down_w, block_expert):
    pt, d = xs.shape
    e, di, _ = gate_w.shape
    nb = pt // BLK
    it = di // DI_T
    grid_spec = pltpu.PrefetchScalarGridSpec(
        num_scalar_prefetch=1,
        grid=(nb, it),
        in_specs=[
            pl.BlockSpec((BLK, d), lambda i, j, be: (i, 0)),
            pl.BlockSpec((1, DI_T, d), lambda i, j, be: (be[i], j, 0)),
            pl.BlockSpec((1, DI_T, d), lambda i, j, be: (be[i], j, 0)),
            pl.BlockSpec((1, d, DI_T), lambda i, j, be: (be[i], 0, j)),
        ],
        out_specs=pl.BlockSpec((BLK, d), lambda i, j, be: (i, 0)),
    )
    return pl.pallas_call(
        _ffn_body,
        grid_spec=grid_spec,
        out_shape=jax.ShapeDtypeStruct((pt, d), jnp.float32),
    )(block_expert, xs, gate_w, up_w, down_w)


def kernel(x, router_w, router_b, gate_w, up_w, down_w):
    b, s, d = x.shape
    t = b * s
    tk = t * K
    pt = tk + E * BLK
    nb = pt // BLK

    flat_x = x.reshape(t, d)

    experts, ranks, wts, counts = _router(flat_x, router_w, router_b)

    padded = ((counts + BLK - 1) // BLK) * BLK
    pend = jnp.cumsum(padded)
    poff = (pend - padded).astype(jnp.int32)
    block_expert = jnp.minimum(
        jnp.searchsorted(pend, jnp.arange(nb, dtype=jnp.int32) * BLK,
                         side='right'),
        E - 1).astype(jnp.int32)

    dest = poff[experts] + ranks

    p_ar = jnp.arange(tk, dtype=jnp.int32)
    tok_of_pair = TB * (p_ar // PB) + (p_ar % TB)

    src = jnp.zeros((pt,), jnp.int32).at[dest].set(tok_of_pair)
    xs = flat_x[src]

    outs = _grouped_ffn(xs, gate_w, up_w, down_w, block_expert)

    t_ar = jnp.arange(t, dtype=jnp.int32)
    pos0 = PB * (t_ar // TB) + (t_ar % TB)
    pos1 = pos0 + TB
    y = (outs[dest[pos0]] * wts[pos0][:, None]
         + outs[dest[pos1]] * wts[pos1][:, None])
    return y.reshape(b, s, d)
